# baseline (speedup 1.0000x reference)
import jax
import jax.numpy as jnp
from jax.experimental import pallas as pl
from jax.experimental.pallas import tpu as pltpu

_CORES = 1
_CHUNK = 8


def _gelu(t):
    return 0.5 * t * (1.0 + jax.lax.erf(t * 0.7071067811865476))


def _expert_body(x_ref, Wf_ref, Wf2_ref, Wf3_ref, Wf4_ref, bf_ref, Wa1_ref,
                 Wa1b_ref, ba1_ref, Wa2_ref, ba2_ref, Wl_ref, bl_ref,
                 out_ref, pred_ref):
    j = pl.program_id(1)
    xb = x_ref[...].astype(jnp.bfloat16)
    q = _CHUNK // 4
    h = _CHUNK // 2
    wfs = (Wf_ref, Wf2_ref, Wf3_ref, Wf4_ref)
    msum = None
    for c in range(_CHUNK):
        src = wfs[c // q][c % q]
        wa1 = Wa1_ref[c] if c < h else Wa1b_ref[c - h]
        wf = src.astype(jnp.bfloat16)
        feat = jnp.dot(xb, wf, preferred_element_type=jnp.float32)
        feat = _gelu(feat + bf_ref[c])
        a1 = jnp.dot(feat.astype(jnp.bfloat16), wa1.astype(jnp.bfloat16),
                     preferred_element_type=jnp.float32)
        a1 = jnp.tanh(a1 + ba1_ref[c])
        s = jnp.sum(a1 * Wa2_ref[c], axis=1, keepdims=True) + ba2_ref[c]
        s = s - jnp.max(s, axis=0, keepdims=True)
        es = jnp.exp(s)
        attn = es / jnp.sum(es, axis=0, keepdims=True)
        M = jnp.sum(attn * feat, axis=0, keepdims=True)
        pred_ref[c] = (jnp.sum(M * Wl_ref[c], axis=1, keepdims=True)
                       + bl_ref[c])
        msum = M if msum is None else msum + M

    @pl.when(j == 0)
    def _init():
        out_ref[...] = jnp.zeros_like(out_ref)

    out_ref[...] += msum


def _reduce_body(part_ref, out_ref):
    out_ref[...] = jnp.sum(part_ref[...], axis=0)


def kernel(x, Wg1, bg1, Wg2, bg2, Wg3, bg3, Wf, bf, Wa1, ba1, Wa2, ba2, Wl, bl):
    E, DIM, L = Wf.shape
    N = x.shape[0]
    D = Wa1.shape[2]
    steps = E // (_CORES * _CHUNK)

    bf3 = bf.reshape(E, 1, L)
    ba13 = ba1.reshape(E, 1, D)
    Wa2r = Wa2.reshape(E, 1, D)
    ba23 = ba2.reshape(E, 1, 1)
    Wlr = Wl.reshape(E, 1, L)
    bl3 = bl.reshape(E, 1, 1)

    parts, preds = pl.pallas_call(
        _expert_body,
        grid=(_CORES, steps),
        in_specs=[
            pl.BlockSpec((N, DIM), lambda i, j: (0, 0)),
            pl.BlockSpec((_CHUNK // 4, DIM, L),
                         lambda i, j: (4 * (i * steps + j), 0, 0)),
            pl.BlockSpec((_CHUNK // 4, DIM, L),
                         lambda i, j: (4 * (i * steps + j) + 1, 0, 0)),
            pl.BlockSpec((_CHUNK // 4, DIM, L),
                         lambda i, j: (4 * (i * steps + j) + 2, 0, 0)),
            pl.BlockSpec((_CHUNK // 4, DIM, L),
                         lambda i, j: (4 * (i * steps + j) + 3, 0, 0)),
            pl.BlockSpec((_CHUNK, 1, L), lambda i, j: (i * steps + j, 0, 0)),
            pl.BlockSpec((_CHUNK // 2, L, D),
                         lambda i, j: (2 * (i * steps + j), 0, 0)),
            pl.BlockSpec((_CHUNK // 2, L, D),
                         lambda i, j: (2 * (i * steps + j) + 1, 0, 0)),
            pl.BlockSpec((_CHUNK, 1, D), lambda i, j: (i * steps + j, 0, 0)),
            pl.BlockSpec((_CHUNK, 1, D), lambda i, j: (i * steps + j, 0, 0)),
            pl.BlockSpec((_CHUNK, 1, 1), lambda i, j: (i * steps + j, 0, 0)),
            pl.BlockSpec((_CHUNK, 1, L), lambda i, j: (i * steps + j, 0, 0)),
            pl.BlockSpec((_CHUNK, 1, 1), lambda i, j: (i * steps + j, 0, 0)),
        ],
        out_specs=[
            pl.BlockSpec((1, 1, L), lambda i, j: (i, 0, 0)),
            pl.BlockSpec((_CHUNK, 1, 1), lambda i, j: (i * steps + j, 0, 0)),
        ],
        out_shape=[
            jax.ShapeDtypeStruct((_CORES, 1, L), jnp.float32),
            jax.ShapeDtypeStruct((E, 1, 1), jnp.float32),
        ],
        compiler_params=pltpu.CompilerParams(
            dimension_semantics=("parallel", "arbitrary")),
    )(x, Wf, Wf, Wf, Wf, bf3, Wa1, Wa1, ba13, Wa2r, ba23, Wlr, bl3)

    out = pl.pallas_call(
        _reduce_body,
        out_shape=jax.ShapeDtypeStruct((1, L), jnp.float32),
    )(parts)
    return out, preds


# speedup vs baseline: 1.0521x; 1.0521x over previous
import jax
import jax.numpy as jnp
from jax.experimental import pallas as pl
from jax.experimental.pallas import tpu as pltpu

_CHUNK = 8


def _gelu(t):
    return 0.5 * t * (1.0 + jax.lax.erf(t * 0.7071067811865476))


def _expert_body(x_ref, Wf_ref, Wf2_ref, bf_ref, Wa1_ref, ba1_ref, Wa2_ref,
                 ba2_ref, Wl_ref, bl_ref, out_ref, pred_ref):
    j = pl.program_id(0)
    xb = x_ref[...].astype(jnp.bfloat16)
    h = _CHUNK // 2
    msum = None
    for c in range(_CHUNK):
        src = Wf_ref[c] if c < h else Wf2_ref[c - h]
        wf = src.astype(jnp.bfloat16)
        feat = jnp.dot(xb, wf, preferred_element_type=jnp.float32)
        feat = _gelu(feat + bf_ref[c])
        a1 = jnp.dot(feat.astype(jnp.bfloat16),
                     Wa1_ref[c].astype(jnp.bfloat16),
                     preferred_element_type=jnp.float32)
        a1 = jnp.tanh(a1 + ba1_ref[c])
        s = jnp.sum(a1 * Wa2_ref[c], axis=1, keepdims=True) + ba2_ref[c]
        s = s - jnp.max(s, axis=0, keepdims=True)
        es = jnp.exp(s)
        attn = es / jnp.sum(es, axis=0, keepdims=True)
        M = jnp.sum(attn * feat, axis=0, keepdims=True)
        pred_ref[c] = (jnp.sum(M * Wl_ref[c], axis=1, keepdims=True)
                       + bl_ref[c])
        msum = M if msum is None else msum + M

    @pl.when(j == 0)
    def _init():
        out_ref[...] = jnp.zeros_like(out_ref)

    out_ref[...] += msum


def kernel(x, Wg1, bg1, Wg2, bg2, Wg3, bg3, Wf, bf, Wa1, ba1, Wa2, ba2, Wl, bl):
    E, DIM, L = Wf.shape
    N = x.shape[0]
    D = Wa1.shape[2]
    steps = E // _CHUNK

    bf3 = bf.reshape(E, 1, L)
    ba13 = ba1.reshape(E, 1, D)
    Wa2r = Wa2.reshape(E, 1, D)
    ba23 = ba2.reshape(E, 1, 1)
    Wlr = Wl.reshape(E, 1, L)
    bl3 = bl.reshape(E, 1, 1)

    out, preds = pl.pallas_call(
        _expert_body,
        grid=(steps,),
        in_specs=[
            pl.BlockSpec((N, DIM), lambda j: (0, 0)),
            pl.BlockSpec((_CHUNK // 2, DIM, L), lambda j: (2 * j, 0, 0)),
            pl.BlockSpec((_CHUNK // 2, DIM, L), lambda j: (2 * j + 1, 0, 0)),
            pl.BlockSpec((_CHUNK, 1, L), lambda j: (j, 0, 0)),
            pl.BlockSpec((_CHUNK, L, D), lambda j: (j, 0, 0)),
            pl.BlockSpec((_CHUNK, 1, D), lambda j: (j, 0, 0)),
            pl.BlockSpec((_CHUNK, 1, D), lambda j: (j, 0, 0)),
            pl.BlockSpec((_CHUNK, 1, 1), lambda j: (j, 0, 0)),
            pl.BlockSpec((_CHUNK, 1, L), lambda j: (j, 0, 0)),
            pl.BlockSpec((_CHUNK, 1, 1), lambda j: (j, 0, 0)),
        ],
        out_specs=[
            pl.BlockSpec((1, L), lambda j: (0, 0)),
            pl.BlockSpec((_CHUNK, 1, 1), lambda j: (j, 0, 0)),
        ],
        out_shape=[
            jax.ShapeDtypeStruct((1, L), jnp.float32),
            jax.ShapeDtypeStruct((E, 1, 1), jnp.float32),
        ],
        compiler_params=pltpu.CompilerParams(
            dimension_semantics=("arbitrary",)),
    )(x, Wf, Wf, bf3, Wa1, ba13, Wa2r, ba23, Wlr, bl3)
    return out, preds


# speedup vs baseline: 1.0544x; 1.0021x over previous
import jax
import jax.numpy as jnp
from jax.experimental import pallas as pl
from jax.experimental.pallas import tpu as pltpu

_CHUNK = 8


def _gelu(t):
    return 0.5 * t * (1.0 + jax.lax.erf(t * 0.7071067811865476))


def _expert_body(x_ref, Wf_ref, Wf2_ref, bf_ref, Wa1_ref, ba1_ref, Wa2_ref,
                 ba2_ref, Wl_ref, bl_ref, out_ref, pred_ref):
    j = pl.program_id(0)
    xb = x_ref[...].astype(jnp.bfloat16)
    h = _CHUNK // 2
    msum = None
    for c in range(_CHUNK):
        src = Wf_ref[c] if c < h else Wf2_ref[c - h]
        wf = src.astype(jnp.bfloat16)
        feat = jnp.dot(xb, wf, preferred_element_type=jnp.float32)
        feat = _gelu(feat + bf_ref[c])
        a1 = jnp.dot(feat.astype(jnp.bfloat16),
                     Wa1_ref[c].astype(jnp.bfloat16),
                     preferred_element_type=jnp.float32)
        a1 = jnp.tanh(a1 + ba1_ref[c])
        s = jnp.sum(a1 * Wa2_ref[c], axis=1, keepdims=True) + ba2_ref[c]
        bound = (jnp.sum(jnp.abs(Wa2_ref[c]), axis=1, keepdims=True)
                 + jnp.abs(ba2_ref[c]))
        es = jnp.exp(s - bound)
        t = es
        n = t.shape[0]
        while n > 8:
            n //= 2
            t = t[:n] + t[n:]
        attn = es / jnp.sum(t, axis=0, keepdims=True)
        M = jnp.sum(attn * feat, axis=0, keepdims=True)
        pred_ref[c] = (jnp.sum(M * Wl_ref[c], axis=1, keepdims=True)
                       + bl_ref[c])
        msum = M if msum is None else msum + M

    @pl.when(j == 0)
    def _init():
        out_ref[...] = jnp.zeros_like(out_ref)

    out_ref[...] += msum


def kernel(x, Wg1, bg1, Wg2, bg2, Wg3, bg3, Wf, bf, Wa1, ba1, Wa2, ba2, Wl, bl):
    E, DIM, L = Wf.shape
    N = x.shape[0]
    D = Wa1.shape[2]
    steps = E // _CHUNK

    bf3 = bf.reshape(E, 1, L)
    ba13 = ba1.reshape(E, 1, D)
    Wa2r = Wa2.reshape(E, 1, D)
    ba23 = ba2.reshape(E, 1, 1)
    Wlr = Wl.reshape(E, 1, L)
    bl3 = bl.reshape(E, 1, 1)

    out, preds = pl.pallas_call(
        _expert_body,
        grid=(steps,),
        in_specs=[
            pl.BlockSpec((N, DIM), lambda j: (0, 0)),
            pl.BlockSpec((_CHUNK // 2, DIM, L), lambda j: (2 * j, 0, 0)),
            pl.BlockSpec((_CHUNK // 2, DIM, L), lambda j: (2 * j + 1, 0, 0)),
            pl.BlockSpec((_CHUNK, 1, L), lambda j: (j, 0, 0)),
            pl.BlockSpec((_CHUNK, L, D), lambda j: (j, 0, 0)),
            pl.BlockSpec((_CHUNK, 1, D), lambda j: (j, 0, 0)),
            pl.BlockSpec((_CHUNK, 1, D), lambda j: (j, 0, 0)),
            pl.BlockSpec((_CHUNK, 1, 1), lambda j: (j, 0, 0)),
            pl.BlockSpec((_CHUNK, 1, L), lambda j: (j, 0, 0)),
            pl.BlockSpec((_CHUNK, 1, 1), lambda j: (j, 0, 0)),
        ],
        out_specs=[
            pl.BlockSpec((1, L), lambda j: (0, 0)),
            pl.BlockSpec((_CHUNK, 1, 1), lambda j: (j, 0, 0)),
        ],
        out_shape=[
            jax.ShapeDtypeStruct((1, L), jnp.float32),
            jax.ShapeDtypeStruct((E, 1, 1), jnp.float32),
        ],
        compiler_params=pltpu.CompilerParams(
            dimension_semantics=("arbitrary",)),
    )(x, Wf, Wf, bf3, Wa1, ba13, Wa2r, ba23, Wlr, bl3)
    return out, preds
